# Initial kernel scaffold; baseline (speedup 1.0000x reference)
import jax
import jax.numpy as jnp
from jax.experimental import pallas as pl


def kernel(x, edge_index, noise, Wl0, Wr0, b0, Wl1, Wr1, b1, Wl2, Wr2, b2):
    raise NotImplementedError("write your pallas kernel here")


# speedup vs baseline: 2.1391x; 2.1391x over previous
import functools

import jax
import jax.numpy as jnp
from jax import lax
from jax.experimental import pallas as pl
from jax.experimental.pallas import tpu as pltpu
from jax.experimental.pallas import tpu_sc as plsc

N = 10000
E = 320000
D = 128
NOISE_RATE = 0.1

NC = 2
NS = 16
CHUNK = 64
NBUF = 2
STEPS = 320
SB = 16
NSTAGE = STEPS // SB
EPT = STEPS * CHUNK
EPAD = NS * EPT
NPAD = N + 112
ZROWS = NPAD // NS
WRB = 632
WRL = N - 15 * WRB


def _make_deg():
  mesh = plsc.VectorSubcoreMesh(
      core_axis_name="c", subcore_axis_name="s",
      num_cores=NC, num_subcores=NS)
  out_type = jax.ShapeDtypeStruct((2 * N, D), jnp.float32)
  scratch = [
      pltpu.VMEM((SB, CHUNK), jnp.int32),
      pltpu.VMEM((CHUNK, D), jnp.float32),
      pltpu.VMEM_SHARED((NPAD, D), jnp.float32),
  ]

  def body(eidx, ones_h, zrows, out, dstv, onesv, dacc):
    c = lax.axis_index("c")
    s = lax.axis_index("s")
    pltpu.sync_copy(zrows, dacc.at[pl.ds(s * ZROWS, ZROWS)])
    pltpu.sync_copy(ones_h, onesv)
    plsc.subcore_barrier()

    def stage(j, carry):
      pltpu.sync_copy(eidx.at[c, 1, s, pl.ds(j * SB, SB)], dstv)

      def inner(i, carry2):
        pltpu.sync_copy(onesv, dacc.at[dstv.at[i]], add=True)
        return carry2

      lax.fori_loop(0, SB, inner, 0)
      return carry

    lax.fori_loop(0, NSTAGE, stage, 0)
    plsc.subcore_barrier()

    @pl.when(s < NS - 1)
    def _():
      pltpu.sync_copy(dacc.at[pl.ds(s * WRB, WRB)],
                      out.at[pl.ds(c * N + s * WRB, WRB)])

    @pl.when(s == NS - 1)
    def _():
      pltpu.sync_copy(dacc.at[pl.ds(15 * WRB, WRL)],
                      out.at[pl.ds(c * N + 15 * WRB, WRL)])

  return pl.kernel(body, out_type=out_type, mesh=mesh, scratch_types=scratch)


def _make_agg():
  mesh = plsc.VectorSubcoreMesh(
      core_axis_name="c", subcore_axis_name="s",
      num_cores=NC, num_subcores=NS)
  out_type = jax.ShapeDtypeStruct((2 * N, D), jnp.float32)
  scratch = [
      pltpu.VMEM((SB, CHUNK), jnp.int32),
      pltpu.VMEM((SB, CHUNK), jnp.int32),
      pltpu.VMEM((NBUF, CHUNK, D), jnp.float32),
      pltpu.VMEM_SHARED((NPAD, D), jnp.float32),
      pltpu.SemaphoreType.DMA,
      pltpu.SemaphoreType.DMA,
  ]

  def body(eidx, x, zrows, out, srcv, dstv, gbuf, acc, sem0, sem1):
    sems = (sem0, sem1)
    c = lax.axis_index("c")
    s = lax.axis_index("s")

    pltpu.sync_copy(zrows, acc.at[pl.ds(s * ZROWS, ZROWS)])
    plsc.subcore_barrier()

    def stage(j, carry):
      pltpu.sync_copy(eidx.at[c, 0, s, pl.ds(j * SB, SB)], srcv)
      pltpu.sync_copy(eidx.at[c, 1, s, pl.ds(j * SB, SB)], dstv)

      def inner(i, carry2):
        pltpu.async_copy(x.at[srcv.at[i]], gbuf.at[0], sems[0]).wait()
        pltpu.sync_copy(gbuf.at[0], acc.at[dstv.at[i]], add=True)
        return carry2

      lax.fori_loop(0, SB, inner, 0)
      return carry

    lax.fori_loop(0, NSTAGE, stage, 0)
    plsc.subcore_barrier()

    @pl.when(s < NS - 1)
    def _():
      pltpu.sync_copy(acc.at[pl.ds(s * WRB, WRB)],
                      out.at[pl.ds(c * N + s * WRB, WRB)])

    @pl.when(s == NS - 1)
    def _():
      pltpu.sync_copy(acc.at[pl.ds(15 * WRB, WRL)],
                      out.at[pl.ds(c * N + 15 * WRB, WRL)])

  return pl.kernel(body, out_type=out_type, mesh=mesh, scratch_types=scratch)


_deg = _make_deg()
_agg = _make_agg()


ROWS = 1000
NB_TC = (2 * N) // ROWS
NB_X0 = N // ROWS


def _x0_body(x_ref, n_ref, o_ref):
  i = pl.program_id(0)
  xv = x_ref[...]

  @pl.when(i < NB_X0)
  def _():
    o_ref[...] = xv

  @pl.when(i >= NB_X0)
  def _():
    nv = n_ref[...]
    nrm = jnp.sqrt(jnp.sum(nv * nv, axis=1, keepdims=True))
    nn = nv / jnp.maximum(nrm, 1e-12)
    o_ref[...] = xv + jnp.sign(xv) * nn * NOISE_RATE


def _stack_noisy(x, noise):
  return pl.pallas_call(
      _x0_body,
      grid=(2 * NB_X0,),
      in_specs=[
          pl.BlockSpec((ROWS, D), lambda i: (i % NB_X0, 0)),
          pl.BlockSpec((ROWS, D), lambda i: (i % NB_X0, 0)),
      ],
      out_specs=pl.BlockSpec((ROWS, D), lambda i: (i, 0)),
      out_shape=jax.ShapeDtypeStruct((2 * N, D), jnp.float32),
  )(x, noise)


def _layer_body(agg_ref, deg_ref, x_ref, wl_ref, wr_ref, b_ref, o_ref):
  deg = jnp.maximum(deg_ref[...][:, 0:1], 1.0)
  mean = agg_ref[...] / deg
  y = (jnp.dot(mean, wl_ref[...], preferred_element_type=jnp.float32)
       + jnp.dot(x_ref[...], wr_ref[...], preferred_element_type=jnp.float32)
       + b_ref[...])
  o_ref[...] = jnp.maximum(y, 0.0)


def _final_body(agg_ref, deg_ref, x_ref, wl_ref, wr_ref, b_ref,
                o_ref, ls_ref):
  deg = jnp.maximum(deg_ref[...][:, 0:1], 1.0)
  mean = agg_ref[...] / deg
  y = (jnp.dot(mean, wl_ref[...], preferred_element_type=jnp.float32)
       + jnp.dot(x_ref[...], wr_ref[...], preferred_element_type=jnp.float32)
       + b_ref[...])
  o_ref[...] = y
  m = jnp.max(y, axis=1, keepdims=True)
  ls_ref[...] = y - m - jnp.log(jnp.sum(jnp.exp(y - m), axis=1,
                                        keepdims=True))


_TC_IN_SPECS = [
    pl.BlockSpec((ROWS, D), lambda i: (i, 0)),
    pl.BlockSpec((ROWS, D), lambda i: (i, 0)),
    pl.BlockSpec((ROWS, D), lambda i: (i, 0)),
    pl.BlockSpec((D, D), lambda i: (0, 0)),
    pl.BlockSpec((D, D), lambda i: (0, 0)),
    pl.BlockSpec((1, D), lambda i: (0, 0)),
]


def _layer(agg, deg, x, wl, wr, b):
  return pl.pallas_call(
      _layer_body,
      grid=(NB_TC,),
      in_specs=_TC_IN_SPECS,
      out_specs=pl.BlockSpec((ROWS, D), lambda i: (i, 0)),
      out_shape=jax.ShapeDtypeStruct((2 * N, D), jnp.float32),
  )(agg, deg, x, wl, wr, b.reshape(1, D))


def _final(agg, deg, x, wl, wr, b):
  return pl.pallas_call(
      _final_body,
      grid=(NB_TC,),
      in_specs=_TC_IN_SPECS,
      out_specs=[pl.BlockSpec((ROWS, D), lambda i: (i, 0)),
                 pl.BlockSpec((ROWS, D), lambda i: (i, 0))],
      out_shape=[jax.ShapeDtypeStruct((2 * N, D), jnp.float32),
                 jax.ShapeDtypeStruct((2 * N, D), jnp.float32)],
  )(agg, deg, x, wl, wr, b.reshape(1, D))


def kernel(x, edge_index, noise, Wl0, Wr0, b0, Wl1, Wr1, b1, Wl2, Wr2, b2):
  src = edge_index[0]
  dst = edge_index[1]
  pad = EPAD - E
  srcp = jnp.concatenate([src, jnp.zeros((pad,), jnp.int32)])
  dstp = jnp.concatenate([dst, jnp.full((pad,), N, jnp.int32)])
  s0 = srcp.reshape(NS, STEPS, CHUNK)
  s1 = (srcp + N).reshape(NS, STEPS, CHUNK)
  dd = dstp.reshape(NS, STEPS, CHUNK)
  eidx = jnp.stack([jnp.stack([s0, dd]), jnp.stack([s1, dd])])

  zrows = jnp.zeros((ZROWS, D), jnp.float32)
  ones_h = jnp.ones((CHUNK, D), jnp.float32)

  x0 = _stack_noisy(x, noise)
  deg = _deg(eidx, ones_h, zrows)
  agg0 = _agg(eidx, x0, zrows)
  x1 = _layer(agg0, deg, x0, Wl0, Wr0, b0)
  agg1 = _agg(eidx, x1, zrows)
  x2 = _layer(agg1, deg, x1, Wl1, Wr1, b1)
  agg2 = _agg(eidx, x2, zrows)
  xo, ls = _final(agg2, deg, x2, Wl2, Wr2, b2)

  return (x2[:N], ls[:N], xo[:N], x2[N:], ls[N:], xo[N:])


# speedup vs baseline: 2.6788x; 1.2523x over previous
import functools

import jax
import jax.numpy as jnp
from jax import lax
from jax.experimental import pallas as pl
from jax.experimental.pallas import tpu as pltpu
from jax.experimental.pallas import tpu_sc as plsc

N = 10000
E = 320000
D = 128
NOISE_RATE = 0.1

NC = 2
NS = 16
CHUNK = 128
NBUF = 2
STEPS = 160
SB = 16
NSTAGE = STEPS // SB
EPT = STEPS * CHUNK
EPAD = NS * EPT
NPAD = N + 112
ZROWS = NPAD // NS
WRB = 632
WRL = N - 15 * WRB


def _make_deg():
  mesh = plsc.VectorSubcoreMesh(
      core_axis_name="c", subcore_axis_name="s",
      num_cores=NC, num_subcores=NS)
  out_type = jax.ShapeDtypeStruct((2 * N, D), jnp.float32)
  scratch = [
      pltpu.VMEM((SB, CHUNK), jnp.int32),
      pltpu.VMEM((CHUNK, D), jnp.float32),
      pltpu.VMEM_SHARED((NPAD, D), jnp.float32),
  ]

  def body(eidx, ones_h, zrows, out, dstv, onesv, dacc):
    c = lax.axis_index("c")
    s = lax.axis_index("s")
    pltpu.sync_copy(zrows, dacc.at[pl.ds(s * ZROWS, ZROWS)])
    pltpu.sync_copy(ones_h, onesv)
    plsc.subcore_barrier()

    def stage(j, carry):
      pltpu.sync_copy(eidx.at[c, 1, s, pl.ds(j * SB, SB)], dstv)

      def inner(i, carry2):
        pltpu.sync_copy(onesv, dacc.at[dstv.at[i]], add=True)
        return carry2

      lax.fori_loop(0, SB, inner, 0)
      return carry

    lax.fori_loop(0, NSTAGE, stage, 0)
    plsc.subcore_barrier()

    @pl.when(s < NS - 1)
    def _():
      pltpu.sync_copy(dacc.at[pl.ds(s * WRB, WRB)],
                      out.at[pl.ds(c * N + s * WRB, WRB)])

    @pl.when(s == NS - 1)
    def _():
      pltpu.sync_copy(dacc.at[pl.ds(15 * WRB, WRL)],
                      out.at[pl.ds(c * N + 15 * WRB, WRL)])

  return pl.kernel(body, out_type=out_type, mesh=mesh, scratch_types=scratch)


def _make_agg():
  mesh = plsc.VectorSubcoreMesh(
      core_axis_name="c", subcore_axis_name="s",
      num_cores=NC, num_subcores=NS)
  out_type = jax.ShapeDtypeStruct((2 * N, D), jnp.float32)
  scratch = [
      pltpu.VMEM((SB, CHUNK), jnp.int32),
      pltpu.VMEM((SB, CHUNK), jnp.int32),
      pltpu.VMEM((NBUF, CHUNK, D), jnp.float32),
      pltpu.VMEM_SHARED((NPAD, D), jnp.float32),
      pltpu.SemaphoreType.DMA,
      pltpu.SemaphoreType.DMA,
  ]

  def body(eidx, x, zrows, out, srcv, dstv, gbuf, acc, sem0, sem1):
    sems = (sem0, sem1)
    c = lax.axis_index("c")
    s = lax.axis_index("s")

    pltpu.sync_copy(zrows, acc.at[pl.ds(s * ZROWS, ZROWS)])
    plsc.subcore_barrier()

    def stage(j, carry):
      pltpu.sync_copy(eidx.at[c, 0, s, pl.ds(j * SB, SB)], srcv)
      pltpu.sync_copy(eidx.at[c, 1, s, pl.ds(j * SB, SB)], dstv)
      for b in range(NBUF):
        pltpu.async_copy(x.at[srcv.at[b]], gbuf.at[b], sems[b])

      def inner(k, carry2):
        i0 = k * NBUF
        for b in range(NBUF):
          i = i0 + b
          pltpu.make_async_copy(x.at[srcv.at[i]], gbuf.at[b], sems[b]).wait()
          pltpu.sync_copy(gbuf.at[b], acc.at[dstv.at[i]], add=True)
          nxt = i + NBUF

          @pl.when(nxt < SB)
          def _():
            pltpu.async_copy(x.at[srcv.at[nxt]], gbuf.at[b], sems[b])
        return carry2

      lax.fori_loop(0, SB // NBUF, inner, 0)
      return carry

    lax.fori_loop(0, NSTAGE, stage, 0)
    plsc.subcore_barrier()

    @pl.when(s < NS - 1)
    def _():
      pltpu.sync_copy(acc.at[pl.ds(s * WRB, WRB)],
                      out.at[pl.ds(c * N + s * WRB, WRB)])

    @pl.when(s == NS - 1)
    def _():
      pltpu.sync_copy(acc.at[pl.ds(15 * WRB, WRL)],
                      out.at[pl.ds(c * N + 15 * WRB, WRL)])

  return pl.kernel(body, out_type=out_type, mesh=mesh, scratch_types=scratch)


_deg = _make_deg()
_agg = _make_agg()


ROWS = 1000
NB_TC = (2 * N) // ROWS
NB_X0 = N // ROWS


def _x0_body(x_ref, n_ref, o_ref):
  i = pl.program_id(0)
  xv = x_ref[...]

  @pl.when(i < NB_X0)
  def _():
    o_ref[...] = xv

  @pl.when(i >= NB_X0)
  def _():
    nv = n_ref[...]
    nrm = jnp.sqrt(jnp.sum(nv * nv, axis=1, keepdims=True))
    nn = nv / jnp.maximum(nrm, 1e-12)
    o_ref[...] = xv + jnp.sign(xv) * nn * NOISE_RATE


def _stack_noisy(x, noise):
  return pl.pallas_call(
      _x0_body,
      grid=(2 * NB_X0,),
      in_specs=[
          pl.BlockSpec((ROWS, D), lambda i: (i % NB_X0, 0)),
          pl.BlockSpec((ROWS, D), lambda i: (i % NB_X0, 0)),
      ],
      out_specs=pl.BlockSpec((ROWS, D), lambda i: (i, 0)),
      out_shape=jax.ShapeDtypeStruct((2 * N, D), jnp.float32),
  )(x, noise)


def _layer_body(agg_ref, deg_ref, x_ref, wl_ref, wr_ref, b_ref, o_ref):
  deg = jnp.maximum(deg_ref[...][:, 0:1], 1.0)
  mean = agg_ref[...] / deg
  y = (jnp.dot(mean, wl_ref[...], preferred_element_type=jnp.float32)
       + jnp.dot(x_ref[...], wr_ref[...], preferred_element_type=jnp.float32)
       + b_ref[...])
  o_ref[...] = jnp.maximum(y, 0.0)


def _final_body(agg_ref, deg_ref, x_ref, wl_ref, wr_ref, b_ref,
                o_ref, ls_ref):
  deg = jnp.maximum(deg_ref[...][:, 0:1], 1.0)
  mean = agg_ref[...] / deg
  y = (jnp.dot(mean, wl_ref[...], preferred_element_type=jnp.float32)
       + jnp.dot(x_ref[...], wr_ref[...], preferred_element_type=jnp.float32)
       + b_ref[...])
  o_ref[...] = y
  m = jnp.max(y, axis=1, keepdims=True)
  ls_ref[...] = y - m - jnp.log(jnp.sum(jnp.exp(y - m), axis=1,
                                        keepdims=True))


_TC_IN_SPECS = [
    pl.BlockSpec((ROWS, D), lambda i: (i, 0)),
    pl.BlockSpec((ROWS, D), lambda i: (i, 0)),
    pl.BlockSpec((ROWS, D), lambda i: (i, 0)),
    pl.BlockSpec((D, D), lambda i: (0, 0)),
    pl.BlockSpec((D, D), lambda i: (0, 0)),
    pl.BlockSpec((1, D), lambda i: (0, 0)),
]


def _layer(agg, deg, x, wl, wr, b):
  return pl.pallas_call(
      _layer_body,
      grid=(NB_TC,),
      in_specs=_TC_IN_SPECS,
      out_specs=pl.BlockSpec((ROWS, D), lambda i: (i, 0)),
      out_shape=jax.ShapeDtypeStruct((2 * N, D), jnp.float32),
  )(agg, deg, x, wl, wr, b.reshape(1, D))


def _final(agg, deg, x, wl, wr, b):
  return pl.pallas_call(
      _final_body,
      grid=(NB_TC,),
      in_specs=_TC_IN_SPECS,
      out_specs=[pl.BlockSpec((ROWS, D), lambda i: (i, 0)),
                 pl.BlockSpec((ROWS, D), lambda i: (i, 0))],
      out_shape=[jax.ShapeDtypeStruct((2 * N, D), jnp.float32),
                 jax.ShapeDtypeStruct((2 * N, D), jnp.float32)],
  )(agg, deg, x, wl, wr, b.reshape(1, D))


def kernel(x, edge_index, noise, Wl0, Wr0, b0, Wl1, Wr1, b1, Wl2, Wr2, b2):
  src = edge_index[0]
  dst = edge_index[1]
  pad = EPAD - E
  srcp = jnp.concatenate([src, jnp.zeros((pad,), jnp.int32)])
  dstp = jnp.concatenate([dst, jnp.full((pad,), N, jnp.int32)])
  s0 = srcp.reshape(NS, STEPS, CHUNK)
  s1 = (srcp + N).reshape(NS, STEPS, CHUNK)
  dd = dstp.reshape(NS, STEPS, CHUNK)
  eidx = jnp.stack([jnp.stack([s0, dd]), jnp.stack([s1, dd])])

  zrows = jnp.zeros((ZROWS, D), jnp.float32)
  ones_h = jnp.ones((CHUNK, D), jnp.float32)

  x0 = _stack_noisy(x, noise)
  deg = _deg(eidx, ones_h, zrows)
  agg0 = _agg(eidx, x0, zrows)
  x1 = _layer(agg0, deg, x0, Wl0, Wr0, b0)
  agg1 = _agg(eidx, x1, zrows)
  x2 = _layer(agg1, deg, x1, Wl1, Wr1, b1)
  agg2 = _agg(eidx, x2, zrows)
  xo, ls = _final(agg2, deg, x2, Wl2, Wr2, b2)

  return (x2[:N], ls[:N], xo[:N], x2[N:], ls[N:], xo[N:])


# speedup vs baseline: 2.7991x; 1.0449x over previous
import functools

import jax
import jax.numpy as jnp
from jax import lax
from jax.experimental import pallas as pl
from jax.experimental.pallas import tpu as pltpu
from jax.experimental.pallas import tpu_sc as plsc

N = 10000
E = 320000
D = 128
NOISE_RATE = 0.1

NC = 2
NS = 16
CHUNK = 64
NBUF = 4
STEPS = 320
SB = 16
NSTAGE = STEPS // SB
EPT = STEPS * CHUNK
EPAD = NS * EPT
NPAD = N + 112
ZROWS = NPAD // NS
WRB = 632
WRL = N - 15 * WRB


def _make_deg():
  mesh = plsc.VectorSubcoreMesh(
      core_axis_name="c", subcore_axis_name="s",
      num_cores=NC, num_subcores=NS)
  out_type = jax.ShapeDtypeStruct((2 * N, D), jnp.float32)
  scratch = [
      pltpu.VMEM((SB, CHUNK), jnp.int32),
      pltpu.VMEM((CHUNK, D), jnp.float32),
      pltpu.VMEM_SHARED((NPAD, D), jnp.float32),
  ]

  def body(eidx, ones_h, zrows, out, dstv, onesv, dacc):
    c = lax.axis_index("c")
    s = lax.axis_index("s")
    pltpu.sync_copy(zrows, dacc.at[pl.ds(s * ZROWS, ZROWS)])
    pltpu.sync_copy(ones_h, onesv)
    plsc.subcore_barrier()

    def stage(j, carry):
      pltpu.sync_copy(eidx.at[c, 1, s, pl.ds(j * SB, SB)], dstv)

      def inner(i, carry2):
        pltpu.sync_copy(onesv, dacc.at[dstv.at[i]], add=True)
        return carry2

      lax.fori_loop(0, SB, inner, 0)
      return carry

    lax.fori_loop(0, NSTAGE, stage, 0)
    plsc.subcore_barrier()

    @pl.when(s < NS - 1)
    def _():
      pltpu.sync_copy(dacc.at[pl.ds(s * WRB, WRB)],
                      out.at[pl.ds(c * N + s * WRB, WRB)])

    @pl.when(s == NS - 1)
    def _():
      pltpu.sync_copy(dacc.at[pl.ds(15 * WRB, WRL)],
                      out.at[pl.ds(c * N + 15 * WRB, WRL)])

  return pl.kernel(body, out_type=out_type, mesh=mesh, scratch_types=scratch)


def _make_agg():
  mesh = plsc.VectorSubcoreMesh(
      core_axis_name="c", subcore_axis_name="s",
      num_cores=NC, num_subcores=NS)
  out_type = jax.ShapeDtypeStruct((2 * N, D), jnp.float32)
  scratch = [
      pltpu.VMEM((SB, CHUNK), jnp.int32),
      pltpu.VMEM((SB, CHUNK), jnp.int32),
      pltpu.VMEM((NBUF, CHUNK, D), jnp.float32),
      pltpu.VMEM_SHARED((NPAD, D), jnp.float32),
      pltpu.SemaphoreType.DMA,
      pltpu.SemaphoreType.DMA,
      pltpu.SemaphoreType.DMA,
      pltpu.SemaphoreType.DMA,
  ]

  def body(eidx, x, zrows, out, srcv, dstv, gbuf, acc, *sems):
    c = lax.axis_index("c")
    s = lax.axis_index("s")

    pltpu.sync_copy(zrows, acc.at[pl.ds(s * ZROWS, ZROWS)])
    plsc.subcore_barrier()

    def stage(j, carry):
      pltpu.sync_copy(eidx.at[c, 0, s, pl.ds(j * SB, SB)], srcv)
      pltpu.sync_copy(eidx.at[c, 1, s, pl.ds(j * SB, SB)], dstv)
      for b in range(NBUF):
        pltpu.async_copy(x.at[srcv.at[b]], gbuf.at[b], sems[b])

      def inner(k, carry2):
        i0 = k * NBUF
        for b in range(NBUF):
          i = i0 + b
          pltpu.make_async_copy(x.at[srcv.at[i]], gbuf.at[b], sems[b]).wait()
          pltpu.sync_copy(gbuf.at[b], acc.at[dstv.at[i]], add=True)
          nxt = i + NBUF

          @pl.when(nxt < SB)
          def _():
            pltpu.async_copy(x.at[srcv.at[nxt]], gbuf.at[b], sems[b])
        return carry2

      lax.fori_loop(0, SB // NBUF, inner, 0)
      return carry

    lax.fori_loop(0, NSTAGE, stage, 0)
    plsc.subcore_barrier()

    @pl.when(s < NS - 1)
    def _():
      pltpu.sync_copy(acc.at[pl.ds(s * WRB, WRB)],
                      out.at[pl.ds(c * N + s * WRB, WRB)])

    @pl.when(s == NS - 1)
    def _():
      pltpu.sync_copy(acc.at[pl.ds(15 * WRB, WRL)],
                      out.at[pl.ds(c * N + 15 * WRB, WRL)])

  return pl.kernel(body, out_type=out_type, mesh=mesh, scratch_types=scratch)


_deg = _make_deg()
_agg = _make_agg()


ROWS = 1000
NB_TC = (2 * N) // ROWS
NB_X0 = N // ROWS


def _x0_body(x_ref, n_ref, o_ref):
  i = pl.program_id(0)
  xv = x_ref[...]

  @pl.when(i < NB_X0)
  def _():
    o_ref[...] = xv

  @pl.when(i >= NB_X0)
  def _():
    nv = n_ref[...]
    nrm = jnp.sqrt(jnp.sum(nv * nv, axis=1, keepdims=True))
    nn = nv / jnp.maximum(nrm, 1e-12)
    o_ref[...] = xv + jnp.sign(xv) * nn * NOISE_RATE


def _stack_noisy(x, noise):
  return pl.pallas_call(
      _x0_body,
      grid=(2 * NB_X0,),
      in_specs=[
          pl.BlockSpec((ROWS, D), lambda i: (i % NB_X0, 0)),
          pl.BlockSpec((ROWS, D), lambda i: (i % NB_X0, 0)),
      ],
      out_specs=pl.BlockSpec((ROWS, D), lambda i: (i, 0)),
      out_shape=jax.ShapeDtypeStruct((2 * N, D), jnp.float32),
  )(x, noise)


def _layer_body(agg_ref, deg_ref, x_ref, wl_ref, wr_ref, b_ref, o_ref):
  deg = jnp.maximum(deg_ref[...][:, 0:1], 1.0)
  mean = agg_ref[...] / deg
  y = (jnp.dot(mean, wl_ref[...], preferred_element_type=jnp.float32)
       + jnp.dot(x_ref[...], wr_ref[...], preferred_element_type=jnp.float32)
       + b_ref[...])
  o_ref[...] = jnp.maximum(y, 0.0)


def _final_body(agg_ref, deg_ref, x_ref, wl_ref, wr_ref, b_ref,
                o_ref, ls_ref):
  deg = jnp.maximum(deg_ref[...][:, 0:1], 1.0)
  mean = agg_ref[...] / deg
  y = (jnp.dot(mean, wl_ref[...], preferred_element_type=jnp.float32)
       + jnp.dot(x_ref[...], wr_ref[...], preferred_element_type=jnp.float32)
       + b_ref[...])
  o_ref[...] = y
  m = jnp.max(y, axis=1, keepdims=True)
  ls_ref[...] = y - m - jnp.log(jnp.sum(jnp.exp(y - m), axis=1,
                                        keepdims=True))


_TC_IN_SPECS = [
    pl.BlockSpec((ROWS, D), lambda i: (i, 0)),
    pl.BlockSpec((ROWS, D), lambda i: (i, 0)),
    pl.BlockSpec((ROWS, D), lambda i: (i, 0)),
    pl.BlockSpec((D, D), lambda i: (0, 0)),
    pl.BlockSpec((D, D), lambda i: (0, 0)),
    pl.BlockSpec((1, D), lambda i: (0, 0)),
]


def _layer(agg, deg, x, wl, wr, b):
  return pl.pallas_call(
      _layer_body,
      grid=(NB_TC,),
      in_specs=_TC_IN_SPECS,
      out_specs=pl.BlockSpec((ROWS, D), lambda i: (i, 0)),
      out_shape=jax.ShapeDtypeStruct((2 * N, D), jnp.float32),
  )(agg, deg, x, wl, wr, b.reshape(1, D))


def _final(agg, deg, x, wl, wr, b):
  return pl.pallas_call(
      _final_body,
      grid=(NB_TC,),
      in_specs=_TC_IN_SPECS,
      out_specs=[pl.BlockSpec((ROWS, D), lambda i: (i, 0)),
                 pl.BlockSpec((ROWS, D), lambda i: (i, 0))],
      out_shape=[jax.ShapeDtypeStruct((2 * N, D), jnp.float32),
                 jax.ShapeDtypeStruct((2 * N, D), jnp.float32)],
  )(agg, deg, x, wl, wr, b.reshape(1, D))


def kernel(x, edge_index, noise, Wl0, Wr0, b0, Wl1, Wr1, b1, Wl2, Wr2, b2):
  src = edge_index[0]
  dst = edge_index[1]
  pad = EPAD - E
  srcp = jnp.concatenate([src, jnp.zeros((pad,), jnp.int32)])
  dstp = jnp.concatenate([dst, jnp.full((pad,), N, jnp.int32)])
  s0 = srcp.reshape(NS, STEPS, CHUNK)
  s1 = (srcp + N).reshape(NS, STEPS, CHUNK)
  dd = dstp.reshape(NS, STEPS, CHUNK)
  eidx = jnp.stack([jnp.stack([s0, dd]), jnp.stack([s1, dd])])

  zrows = jnp.zeros((ZROWS, D), jnp.float32)
  ones_h = jnp.ones((CHUNK, D), jnp.float32)

  x0 = _stack_noisy(x, noise)
  deg = _deg(eidx, ones_h, zrows)
  agg0 = _agg(eidx, x0, zrows)
  x1 = _layer(agg0, deg, x0, Wl0, Wr0, b0)
  agg1 = _agg(eidx, x1, zrows)
  x2 = _layer(agg1, deg, x1, Wl1, Wr1, b1)
  agg2 = _agg(eidx, x2, zrows)
  xo, ls = _final(agg2, deg, x2, Wl2, Wr2, b2)

  return (x2[:N], ls[:N], xo[:N], x2[N:], ls[N:], xo[N:])
